# baseline (speedup 1.0000x reference)
import jax
import jax.numpy as jnp
from jax.experimental import pallas as pl
from jax.experimental.pallas import tpu as pltpu

_N = 10000
_F = 128
_BM = 80
_NV = 3
_NS = 6
_STEPS = (_N // _BM) * _NV


def _fused_body(x1_ref, x2_ref, x3_ref, w1_ref, w1s_ref, w2s_ref, w3s_ref,
                bias_ref, xb_ref, a1_hbm, a2_hbm, a3_hbm, out_ref,
                s1_ref, s2_ref, s3_ref, buf_ref, sem_ref):
    i = pl.program_id(0)

    def issue(j):
        m = j // _NV
        v = j % _NV
        slot = j % _NS
        dst = buf_ref.at[slot]
        sem = sem_ref.at[slot]

        @pl.when(v == 0)
        def _():
            pltpu.make_async_copy(
                a1_hbm.at[pl.ds(m * _BM, _BM), :], dst, sem).start()

        @pl.when(v == 1)
        def _():
            pltpu.make_async_copy(
                a2_hbm.at[pl.ds(m * _BM, _BM), :], dst, sem).start()

        @pl.when(v == 2)
        def _():
            pltpu.make_async_copy(
                a3_hbm.at[pl.ds(m * _BM, _BM), :], dst, sem).start()

    @pl.when(i == 0)
    def _():
        def prologue(j, carry):
            issue(j)
            return carry

        jax.lax.fori_loop(0, _NS, prologue, 0)

        cb = 2000

        def chunk(j, carry):
            sl = pl.ds(j * cb, cb)
            s1_ref[sl, :] = jnp.dot(x1_ref[sl, :], w1s_ref[...],
                                    preferred_element_type=jnp.float32,
                                    precision=jax.lax.Precision.HIGHEST)
            s2_ref[sl, :] = jnp.dot(x2_ref[sl, :], w2s_ref[...],
                                    preferred_element_type=jnp.float32,
                                    precision=jax.lax.Precision.HIGHEST)
            s3_ref[sl, :] = jnp.dot(x3_ref[sl, :], w3s_ref[...],
                                    preferred_element_type=jnp.float32,
                                    precision=jax.lax.Precision.HIGHEST)
            return carry

        jax.lax.fori_loop(0, _N // cb, chunk, 0)

    @pl.when(jnp.logical_and(i > 0, i + _NS - 1 < _STEPS))
    def _():
        issue(i + _NS - 1)

    slot = i % _NS
    v = i % _NV
    pltpu.make_async_copy(a1_hbm.at[pl.ds(0, _BM), :], buf_ref.at[slot],
                          sem_ref.at[slot]).wait()
    a = buf_ref[slot]

    @pl.when(v == 0)
    def _():
        base = jnp.dot(xb_ref[...], w1_ref[...],
                       preferred_element_type=jnp.float32,
                       precision=jax.lax.Precision.HIGHEST)
        out_ref[...] = (base + bias_ref[...]
                        + jnp.dot(a, s1_ref[...],
                                  preferred_element_type=jnp.float32,
                                  precision=jax.lax.Precision.DEFAULT))

    @pl.when(v == 1)
    def _():
        out_ref[...] = out_ref[...] + jnp.dot(
            a, s2_ref[...], preferred_element_type=jnp.float32,
            precision=jax.lax.Precision.DEFAULT)

    @pl.when(v == 2)
    def _():
        out_ref[...] = out_ref[...] + jnp.dot(
            a, s3_ref[...], preferred_element_type=jnp.float32,
            precision=jax.lax.Precision.DEFAULT)


def kernel(self_input, self_adj, view2_input, view2_adj, view3_input,
           view3_adj, weight_self, weight_view2, weight_view3,
           weight_all_views, bias):
    c = (1.01 * weight_all_views.astype(jnp.float32)).reshape(3)
    w1s = weight_self * c[0]
    w2s = weight_view2 * c[1]
    w3s = weight_view3 * c[2]
    bias2d = bias.reshape(1, _F).astype(jnp.float32)

    full = pl.BlockSpec((_N, _F), lambda i: (0, 0))
    wspec = pl.BlockSpec((_F, _F), lambda i: (0, 0))
    any_spec = pl.BlockSpec(memory_space=pl.ANY)
    row_spec = pl.BlockSpec((_BM, _F), lambda i: (i // _NV, 0))

    out = pl.pallas_call(
        _fused_body,
        grid=(_STEPS,),
        in_specs=[full, full, full, wspec, wspec, wspec, wspec,
                  pl.BlockSpec((1, _F), lambda i: (0, 0)),
                  row_spec, any_spec, any_spec, any_spec],
        out_specs=row_spec,
        out_shape=jax.ShapeDtypeStruct((_N, _F), jnp.float32),
        scratch_shapes=[pltpu.VMEM((_N, _F), jnp.float32)] * 3
        + [pltpu.VMEM((_NS, _BM, _N), jnp.float32),
           pltpu.SemaphoreType.DMA((_NS,))],
        compiler_params=pltpu.CompilerParams(
            dimension_semantics=("arbitrary",),
        ),
    )(self_input, view2_input, view3_input, weight_self, w1s, w2s, w3s,
      bias2d, self_input, self_adj, view2_adj, view3_adj)

    return out


# speedup vs baseline: 1.0086x; 1.0086x over previous
import jax
import jax.numpy as jnp
from jax.experimental import pallas as pl
from jax.experimental.pallas import tpu as pltpu

_N = 10000
_F = 128
_BM = 80


def _fused_body(x1_ref, x2_ref, x3_ref, w1_ref, w1s_ref, w2s_ref, w3s_ref,
                bias_ref, a1_ref, a2_ref, a3_ref, out_ref,
                s1_ref, s2_ref, s3_ref):
    i = pl.program_id(0)

    @pl.when(i == 0)
    def _():
        cb = 2000

        def chunk(j, carry):
            sl = pl.ds(j * cb, cb)
            s1_ref[sl, :] = jnp.dot(x1_ref[sl, :], w1s_ref[...],
                                    preferred_element_type=jnp.float32,
                                    precision=jax.lax.Precision.HIGHEST)
            s2_ref[sl, :] = jnp.dot(x2_ref[sl, :], w2s_ref[...],
                                    preferred_element_type=jnp.float32,
                                    precision=jax.lax.Precision.HIGHEST)
            s3_ref[sl, :] = jnp.dot(x3_ref[sl, :], w3s_ref[...],
                                    preferred_element_type=jnp.float32,
                                    precision=jax.lax.Precision.HIGHEST)
            return carry

        jax.lax.fori_loop(0, _N // cb, chunk, 0)

    acc = jnp.dot(a1_ref[...], s1_ref[...], preferred_element_type=jnp.float32,
                  precision=jax.lax.Precision.DEFAULT)
    acc = acc + jnp.dot(a2_ref[...], s2_ref[...],
                        preferred_element_type=jnp.float32,
                        precision=jax.lax.Precision.DEFAULT)
    acc = acc + jnp.dot(a3_ref[...], s3_ref[...],
                        preferred_element_type=jnp.float32,
                        precision=jax.lax.Precision.DEFAULT)
    base = jnp.dot(x1_ref[pl.ds(i * _BM, _BM), :], w1_ref[...],
                   preferred_element_type=jnp.float32,
                   precision=jax.lax.Precision.HIGHEST)
    out_ref[...] = acc + base + bias_ref[...]


def kernel(self_input, self_adj, view2_input, view2_adj, view3_input,
           view3_adj, weight_self, weight_view2, weight_view3,
           weight_all_views, bias):
    c = (1.01 * weight_all_views.astype(jnp.float32)).reshape(3)
    w1s = weight_self * c[0]
    w2s = weight_view2 * c[1]
    w3s = weight_view3 * c[2]
    bias2d = bias.reshape(1, _F).astype(jnp.float32)

    full = pl.BlockSpec((_N, _F), lambda i: (0, 0))
    wspec = pl.BlockSpec((_F, _F), lambda i: (0, 0))
    adj_spec = pl.BlockSpec((_BM, _N), lambda i: (i, 0))
    row_spec = pl.BlockSpec((_BM, _F), lambda i: (i, 0))

    out = pl.pallas_call(
        _fused_body,
        grid=(_N // _BM,),
        in_specs=[full, full, full, wspec, wspec, wspec, wspec,
                  pl.BlockSpec((1, _F), lambda i: (0, 0)),
                  adj_spec, adj_spec, adj_spec],
        out_specs=row_spec,
        out_shape=jax.ShapeDtypeStruct((_N, _F), jnp.float32),
        scratch_shapes=[pltpu.VMEM((_N, _F), jnp.float32)] * 3,
        compiler_params=pltpu.CompilerParams(
            dimension_semantics=("arbitrary",),
        ),
    )(self_input, view2_input, view3_input, weight_self, w1s, w2s, w3s,
      bias2d, self_adj, view2_adj, view3_adj)

    return out
